# Initial kernel scaffold; baseline (speedup 1.0000x reference)
import jax
import jax.numpy as jnp
from jax.experimental import pallas as pl


def kernel(x, router_W, router_b, sW1, sb1, sW2, sb2, rW1, rb1, rW2, rb2):
    raise NotImplementedError("write your pallas kernel here")


# speedup vs baseline: 1.0590x; 1.0590x over previous
import functools

import jax
import jax.numpy as jnp
from jax import lax
from jax.experimental import pallas as pl
from jax.experimental.pallas import tpu as pltpu
from jax.experimental.pallas import tpu_sc as plsc

T = 4096
H = 1024
ID = 1024
E = 7
EP = 8
BT = 256
NB = T // BT + 8
SLOTS = NB * BT
TM = 512

NC = 2
NS = 16
NW = NC * NS
PER_W = SLOTS // NW
CH = 32
NCH = PER_W // CH


def _gelu(t):
    return 0.5 * t * (1.0 + lax.erf(t * 0.7071067811865476))


def _dotT(a, b):
    return lax.dot_general(a, b, (((1,), (1,)), ((), ())),
                           preferred_element_type=jnp.float32)


def _shared_router_body(x_ref, w1_ref, b1_ref, w2_ref, b2_ref, rw_ref, rb_ref,
                        y0_ref, gate_ref, eidx_ref):
    xb = x_ref[...]
    h = _gelu(_dotT(xb, w1_ref[...]) + b1_ref[...])
    y0 = _dotT(h, w2_ref[...]) + b2_ref[...]
    y0_ref[...] = y0 + xb
    logits = _dotT(xb, rw_ref[...]) + rb_ref[...]
    lanes = lax.broadcasted_iota(jnp.int32, logits.shape, 1)
    logits = jnp.where(lanes < E, logits, -1e30)
    m = jnp.max(logits, axis=1, keepdims=True)
    denom = jnp.sum(jnp.exp(logits - m), axis=1, keepdims=True)
    g = 1.0 / denom
    idx = jnp.min(jnp.where(logits == m, lanes, EP), axis=1, keepdims=True)
    gate_ref[...] = jnp.broadcast_to(g, gate_ref.shape)
    eidx_ref[...] = jnp.broadcast_to(idx, eidx_ref.shape).astype(jnp.int32)


def _shared_and_router(xf, sW1, sb1, sW2, sb2, rWp, rbp):
    return pl.pallas_call(
        _shared_router_body,
        grid=(T // TM,),
        in_specs=[
            pl.BlockSpec((TM, H), lambda i: (i, 0)),
            pl.BlockSpec((ID, H), lambda i: (0, 0)),
            pl.BlockSpec((1, ID), lambda i: (0, 0)),
            pl.BlockSpec((H, ID), lambda i: (0, 0)),
            pl.BlockSpec((1, H), lambda i: (0, 0)),
            pl.BlockSpec((EP, H), lambda i: (0, 0)),
            pl.BlockSpec((1, EP), lambda i: (0, 0)),
        ],
        out_specs=[
            pl.BlockSpec((TM, H), lambda i: (i, 0)),
            pl.BlockSpec((TM, EP), lambda i: (i, 0)),
            pl.BlockSpec((TM, EP), lambda i: (i, 0)),
        ],
        out_shape=[
            jax.ShapeDtypeStruct((T, H), jnp.float32),
            jax.ShapeDtypeStruct((T, EP), jnp.float32),
            jax.ShapeDtypeStruct((T, EP), jnp.int32),
        ],
    )(xf, sW1, sb1, sW2, sb2, rWp, rbp)


def _expert_ffn_body(beids_ref, xp_ref, y0p_ref, g_ref,
                     w1_ref, b1_ref, w2_ref, b2_ref, out_ref):
    xb = xp_ref[...]
    h = _gelu(_dotT(xb, w1_ref[0]) + b1_ref[0])
    y = _dotT(h, w2_ref[0]) + b2_ref[0]
    out_ref[...] = y0p_ref[...] + g_ref[:, :1] * y


def _expert_ffn(beids, Xp, Y0p, gcols, rW1, rb1r, rW2, rb2r):
    grid_spec = pltpu.PrefetchScalarGridSpec(
        num_scalar_prefetch=1,
        grid=(NB,),
        in_specs=[
            pl.BlockSpec((BT, H), lambda i, beids: (i, 0)),
            pl.BlockSpec((BT, H), lambda i, beids: (i, 0)),
            pl.BlockSpec((BT, EP), lambda i, beids: (i, 0)),
            pl.BlockSpec((1, ID, H), lambda i, beids: (beids[i], 0, 0)),
            pl.BlockSpec((1, 1, ID), lambda i, beids: (beids[i], 0, 0)),
            pl.BlockSpec((1, H, ID), lambda i, beids: (beids[i], 0, 0)),
            pl.BlockSpec((1, 1, H), lambda i, beids: (beids[i], 0, 0)),
        ],
        out_specs=pl.BlockSpec((BT, H), lambda i, beids: (i, 0)),
    )
    return pl.pallas_call(
        _expert_ffn_body,
        grid_spec=grid_spec,
        out_shape=jax.ShapeDtypeStruct((SLOTS, H), jnp.float32),
    )(beids, Xp, Y0p, gcols, rW1, rb1r, rW2, rb2r)


def _sc_gather(xf, y0, toks_g):
    mesh = plsc.VectorSubcoreMesh(core_axis_name="c", subcore_axis_name="s")

    @functools.partial(
        pl.kernel,
        mesh=mesh,
        out_type=[jax.ShapeDtypeStruct((SLOTS, H), jnp.float32),
                  jax.ShapeDtypeStruct((SLOTS, H), jnp.float32)],
        scratch_types=[pltpu.VMEM((CH,), jnp.int32),
                       pltpu.VMEM((CH, H), jnp.float32),
                       pltpu.VMEM((CH, H), jnp.float32),
                       pltpu.SemaphoreType.DMA,
                       pltpu.SemaphoreType.DMA],
    )
    def gk(x_hbm, y0_hbm, toks_hbm, xp_hbm, y0p_hbm, idx_v, xr, yr, s1, s2):
        wid = lax.axis_index("s") * NC + lax.axis_index("c")
        for ch in range(NCH):
            base = pl.multiple_of(wid * PER_W + ch * CH, 8)
            pltpu.sync_copy(toks_hbm.at[pl.ds(base, CH)], idx_v)
            c1 = pltpu.async_copy(x_hbm.at[idx_v], xr, s1)
            c2 = pltpu.async_copy(y0_hbm.at[idx_v], yr, s2)
            c1.wait()
            c2.wait()
            pltpu.sync_copy(xr, xp_hbm.at[pl.ds(base, CH)])
            pltpu.sync_copy(yr, y0p_hbm.at[pl.ds(base, CH)])

    return gk(xf, y0, toks_g)


def _sc_scatter(Yp, toks):
    mesh = plsc.VectorSubcoreMesh(core_axis_name="c", subcore_axis_name="s")

    @functools.partial(
        pl.kernel,
        mesh=mesh,
        out_type=jax.ShapeDtypeStruct((T + 8, H), jnp.float32),
        scratch_types=[pltpu.VMEM((CH,), jnp.int32),
                       pltpu.VMEM((CH, H), jnp.float32),
                       pltpu.SemaphoreType.DMA],
    )
    def sk(yp_hbm, toks_hbm, out_hbm, idx_v, rows, sem):
        wid = lax.axis_index("s") * NC + lax.axis_index("c")
        for ch in range(NCH):
            base = pl.multiple_of(wid * PER_W + ch * CH, 8)
            pltpu.sync_copy(toks_hbm.at[pl.ds(base, CH)], idx_v)
            pltpu.sync_copy(yp_hbm.at[pl.ds(base, CH)], rows)
            pltpu.async_copy(rows, out_hbm.at[idx_v], sem).wait()

    return sk(Yp, toks)


def kernel(x, router_W, router_b, sW1, sb1, sW2, sb2, rW1, rb1, rW2, rb2):
    Bb, S, _ = x.shape
    xf = x.reshape(T, H)
    rWp = jnp.zeros((EP, H), jnp.float32).at[:E].set(router_W)
    rbp = jnp.zeros((1, EP), jnp.float32).at[0, :E].set(router_b)

    y0, gate8, eidx8 = _shared_and_router(
        xf, sW1, sb1.reshape(1, ID), sW2, sb2.reshape(1, H), rWp, rbp)

    eid = eidx8[:, 0]
    gate = gate8[:, 0]

    oh = (eid[:, None] == jnp.arange(E, dtype=jnp.int32)[None, :]).astype(jnp.int32)
    csum = jnp.cumsum(oh, axis=0)
    rank = jnp.take_along_axis(csum, eid[:, None], axis=1)[:, 0] - 1
    counts = csum[-1]
    nblk = (counts + BT - 1) // BT
    cnb = jnp.cumsum(nblk)
    pstart = (cnb - nblk) * BT
    slot = pstart[eid] + rank
    toks = jnp.full((SLOTS,), T, jnp.int32).at[slot].set(
        jnp.arange(T, dtype=jnp.int32))
    toks_g = jnp.minimum(toks, T - 1)
    gate_slot = jnp.where(toks < T, gate[toks_g], 0.0)
    gcols = jnp.broadcast_to(gate_slot[:, None], (SLOTS, EP))
    beids = jnp.clip(
        jnp.searchsorted(cnb, jnp.arange(NB, dtype=jnp.int32), side="right"),
        0, E - 1).astype(jnp.int32)

    Xp, Y0p = _sc_gather(xf, y0, toks_g)
    Yp = _expert_ffn(beids, Xp, Y0p, gcols,
                     rW1, rb1.reshape(E, 1, ID), rW2, rb2.reshape(E, 1, H))
    out = _sc_scatter(Yp, toks)
    return out[:T].reshape(Bb, S, H)
